# Initial kernel scaffold; baseline (speedup 1.0000x reference)
import jax
import jax.numpy as jnp
from jax.experimental import pallas as pl


def kernel(input_ids, W_lin, b_lin, W_router, W1, b1, W2, b2):
    raise NotImplementedError("write your pallas kernel here")


# speedup vs baseline: 1.3447x; 1.3447x over previous
import functools

import jax
import jax.numpy as jnp
from jax.experimental import pallas as pl
from jax.experimental.pallas import tpu as pltpu


def _fused_body(x_ref, wl_ref, bl_ref, wr_ref, w1_ref, b1_ref, w2_ref, b2_ref,
                out_ref, *, n_exp):
    x = x_ref[...]
    h = jnp.dot(x, wl_ref[...]) + bl_ref[...][None, :]
    logits = jnp.dot(h, wr_ref[...])
    probs = jax.nn.softmax(logits, axis=-1)
    bt = probs.shape[0]
    iota = jax.lax.broadcasted_iota(jnp.int32, (bt, n_exp), 1)
    m0 = jnp.max(probs, axis=-1, keepdims=True)
    i0 = jnp.min(jnp.where(probs == m0, iota, n_exp), axis=-1, keepdims=True)
    probs1 = jnp.where(iota == i0, -1.0, probs)
    m1 = jnp.max(probs1, axis=-1, keepdims=True)
    i1 = jnp.min(jnp.where(probs1 == m1, iota, n_exp), axis=-1, keepdims=True)
    denom = m0 + m1
    w0 = m0 / denom
    w1 = m1 / denom
    acc = jnp.zeros_like(h)
    for e in range(n_exp):
        gate = jnp.where(i0 == e, w0, 0.0) + jnp.where(i1 == e, w1, 0.0)
        inter = jnp.maximum(jnp.dot(h, w1_ref[e]) + b1_ref[e][None, :], 0.0)
        exp_out = jnp.dot(inter, w2_ref[e]) + b2_ref[e][None, :]
        acc = acc + gate * exp_out
    out_ref[...] = acc


def kernel(input_ids, W_lin, b_lin, W_router, W1, b1, W2, b2):
    B, S, D = input_ids.shape
    E, _, F = W1.shape
    T = B * S
    x = input_ids.reshape(T, D)
    BT = 512
    grid = (T // BT,)
    body = functools.partial(_fused_body, n_exp=E)
    out = pl.pallas_call(
        body,
        grid=grid,
        in_specs=[
            pl.BlockSpec((BT, D), lambda i: (i, 0)),
            pl.BlockSpec((D, D), lambda i: (0, 0)),
            pl.BlockSpec((D,), lambda i: (0,)),
            pl.BlockSpec((D, E), lambda i: (0, 0)),
            pl.BlockSpec((E, D, F), lambda i: (0, 0, 0)),
            pl.BlockSpec((E, F), lambda i: (0, 0)),
            pl.BlockSpec((E, F, D), lambda i: (0, 0, 0)),
            pl.BlockSpec((E, D), lambda i: (0, 0)),
        ],
        out_specs=pl.BlockSpec((BT, D), lambda i: (i, 0)),
        out_shape=jax.ShapeDtypeStruct((T, D), jnp.float32),
        compiler_params=pltpu.CompilerParams(
            dimension_semantics=("arbitrary",),
        ),
    )(x, W_lin, b_lin, W_router, W1, b1, W2, b2)
    return out.reshape(B, S, D)
